# baseline (speedup 1.0000x reference)
import functools

import jax
import jax.numpy as jnp
from jax import lax
from jax.experimental import pallas as pl
from jax.experimental.pallas import tpu as pltpu
from jax.experimental.pallas import tpu_sc as plsc

_L = 16
_BSC = 0
_CBLK = 16


def _sc_body(nBatch, outDim, nWin, nSeq,
             trf_hbm, src_hbm, ov_hbm, out_hbm,
             buf_a, buf_b, src_v, ov_v, acc_v, sem_a, sem_b):
    nLen = 2 * nSeq
    accN = nLen + nWin
    half = nSeq // 2
    cid = lax.axis_index("c")
    sid = lax.axis_index("s")
    wid = sid * 2 + cid
    jobs_per_worker = (nBatch * outDim) // 32
    cblocks = 32 // nBatch
    b = wid // cblocks
    c0 = (wid % cblocks) * jobs_per_worker

    pltpu.sync_copy(src_hbm.at[b], src_v)
    pltpu.sync_copy(ov_hbm, ov_v)
    ovec = ov_v[...]

    def in_copy(c, h, buf, sem):
        return pltpu.make_async_copy(
            trf_hbm.at[b, c, :, pl.ds(h * half, half)], buf, sem)

    def compute(buf, s_base):
        @plsc.parallel_loop(0, half // _L, unroll=2)
        def sgroup(sb):
            tvec = src_v[pl.ds(s_base + sb * _L, _L)]
            vals = [buf[w, pl.ds(sb * _L, _L)] for w in range(nWin)]
            idxs = [tvec + r for r in range(8)]
            for w in range(nWin):
                base = (w // 8) * 8
                plsc.addupdate_scatter(
                    acc_v.at[pl.ds(base, accN - base)], [idxs[w % 8]], vals[w])

    in_copy(c0, 0, buf_a, sem_a).start()

    def job(j, _):
        c = c0 + j

        def init(i, _):
            acc_v[pl.ds(i * _L, _L)] = ovec
            return _
        lax.fori_loop(0, accN // _L, init, None)

        in_copy(c, 0, buf_a, sem_a).wait()
        in_copy(c, 1, buf_b, sem_b).start()
        compute(buf_a, 0)
        in_copy(c, 1, buf_b, sem_b).wait()

        @pl.when(j < jobs_per_worker - 1)
        def _prefetch():
            in_copy(c + 1, 0, buf_a, sem_a).start()

        compute(buf_b, half)
        pltpu.sync_copy(acc_v.at[pl.ds(0, nLen)], out_hbm.at[b, c])
        return _
    lax.fori_loop(0, jobs_per_worker, job, None)


def _run_sc(TRFs, sourceIdx, ov_arr, b_count):
    _, outDim, nWin, nSeq = TRFs.shape
    nLen = 2 * nSeq
    accN = nLen + nWin
    mesh = plsc.VectorSubcoreMesh(core_axis_name="c", subcore_axis_name="s")
    run = pl.kernel(
        functools.partial(_sc_body, b_count, outDim, nWin, nSeq),
        mesh=mesh,
        compiler_params=pltpu.CompilerParams(needs_layout_passes=False),
        out_type=jax.ShapeDtypeStruct((b_count, outDim, nLen), jnp.float32),
        scratch_types=[
            pltpu.VMEM((nWin, nSeq // 2), jnp.float32),
            pltpu.VMEM((nWin, nSeq // 2), jnp.float32),
            pltpu.VMEM((nSeq,), jnp.int32),
            pltpu.VMEM((_L,), jnp.float32),
            pltpu.VMEM((accN,), jnp.float32),
            pltpu.SemaphoreType.DMA,
            pltpu.SemaphoreType.DMA,
        ],
    )
    return run(TRFs, sourceIdx, ov_arr)


def _tc_body(nWin, nSeq, trf_ref, src_ref, ov_ref, oe_ref, oo_ref):
    trf = trf_ref[0]
    src = src_ref[0]
    o = src - 2 * lax.broadcasted_iota(jnp.int32, (1, nSeq), 1)
    m0 = (o == 0).astype(jnp.float32)
    m1 = 1.0 - m0
    pad = nWin // 2

    def shifted(x, left):
        parts = []
        if left:
            parts.append(jnp.zeros((_CBLK, left), jnp.float32))
        parts.append(x)
        if pad - left:
            parts.append(jnp.zeros((_CBLK, pad - left), jnp.float32))
        return jnp.concatenate(parts, axis=1) if len(parts) > 1 else x

    acc_e = jnp.zeros((_CBLK, nSeq + pad), jnp.float32)
    acc_o = jnp.zeros((_CBLK, nSeq + pad), jnp.float32)
    for w in range(nWin):
        v = trf[:, w, :]
        mE, mO = (m0, m1) if w % 2 == 0 else (m1, m0)
        acc_e = acc_e + shifted(v * mE, (w + 1) // 2)
        acc_o = acc_o + shifted(v * mO, w // 2)
    ov = ov_ref[0, 0]
    oe_ref[0] = acc_e[:, :nSeq] + ov
    oo_ref[0] = acc_o[:, :nSeq] + ov


def _run_tc(TRFs, sourceIdx, ov2, b_lo):
    nBatch, outDim, nWin, nSeq = TRFs.shape
    nTc = nBatch - b_lo
    src3 = sourceIdx.reshape(nBatch, 1, nSeq)
    grid = (nTc, outDim // _CBLK)
    oe, oo = pl.pallas_call(
        functools.partial(_tc_body, nWin, nSeq),
        grid=grid,
        in_specs=[
            pl.BlockSpec((1, _CBLK, nWin, nSeq),
                         lambda i, j: (i + b_lo, j, 0, 0)),
            pl.BlockSpec((1, 1, nSeq), lambda i, j: (i + b_lo, 0, 0)),
            pl.BlockSpec((1, 1), lambda i, j: (0, 0)),
        ],
        out_specs=[
            pl.BlockSpec((1, _CBLK, nSeq), lambda i, j: (i, j, 0)),
            pl.BlockSpec((1, _CBLK, nSeq), lambda i, j: (i, j, 0)),
        ],
        out_shape=[
            jax.ShapeDtypeStruct((nTc, outDim, nSeq), jnp.float32),
            jax.ShapeDtypeStruct((nTc, outDim, nSeq), jnp.float32),
        ],
        compiler_params=pltpu.CompilerParams(
            dimension_semantics=("parallel", "parallel")),
    )(TRFs, src3, ov2)
    return jnp.stack([oe, oo], axis=-1).reshape(nTc, outDim, 2 * nSeq)


def kernel(TRFs, sourceIdx, nRealLen):
    nBatch, outDim, nWin, nSeq = TRFs.shape

    maxSrc = jnp.max(sourceIdx[:, -1])
    overflow = jnp.maximum(maxSrc + 1 - nRealLen, 0).astype(jnp.float32)

    out_tc = _run_tc(TRFs, sourceIdx, overflow.reshape(1, 1), _BSC)
    if _BSC == 0:
        return out_tc
    out_sc = _run_sc(TRFs, sourceIdx,
                     jnp.broadcast_to(overflow, (_L,)), _BSC)
    return jnp.concatenate([out_sc, out_tc], axis=0)


# speedup vs baseline: 1.6501x; 1.6501x over previous
import functools

import jax
import jax.numpy as jnp
from jax import lax
from jax.experimental import pallas as pl
from jax.experimental.pallas import tpu as pltpu
from jax.experimental.pallas import tpu_sc as plsc

_L = 16
_BSC = 4
_CBLK = 16


def _sc_body(nBatch, outDim, nWin, nSeq,
             trf_hbm, src_hbm, ov_hbm, out_hbm,
             buf_a, buf_b, src_v, ov_v, acc_v, sem_a, sem_b):
    nLen = 2 * nSeq
    accN = nLen + nWin
    half = nSeq // 2
    cid = lax.axis_index("c")
    sid = lax.axis_index("s")
    wid = sid * 2 + cid
    jobs_per_worker = (nBatch * outDim) // 32
    cblocks = 32 // nBatch
    b = wid // cblocks
    c0 = (wid % cblocks) * jobs_per_worker

    pltpu.sync_copy(src_hbm.at[b], src_v)
    pltpu.sync_copy(ov_hbm, ov_v)
    ovec = ov_v[...]

    def in_copy(c, h, buf, sem):
        return pltpu.make_async_copy(
            trf_hbm.at[b, c, :, pl.ds(h * half, half)], buf, sem)

    def compute(buf, s_base):
        @plsc.parallel_loop(0, half // _L, unroll=2)
        def sgroup(sb):
            tvec = src_v[pl.ds(s_base + sb * _L, _L)]
            vals = [buf[w, pl.ds(sb * _L, _L)] for w in range(nWin)]
            idxs = [tvec + r for r in range(8)]
            for w in range(nWin):
                base = (w // 8) * 8
                plsc.addupdate_scatter(
                    acc_v.at[pl.ds(base, accN - base)], [idxs[w % 8]], vals[w])

    in_copy(c0, 0, buf_a, sem_a).start()

    def job(j, _):
        c = c0 + j

        def init(i, _):
            acc_v[pl.ds(i * _L, _L)] = ovec
            return _
        lax.fori_loop(0, accN // _L, init, None)

        in_copy(c, 0, buf_a, sem_a).wait()
        in_copy(c, 1, buf_b, sem_b).start()
        compute(buf_a, 0)
        in_copy(c, 1, buf_b, sem_b).wait()

        @pl.when(j < jobs_per_worker - 1)
        def _prefetch():
            in_copy(c + 1, 0, buf_a, sem_a).start()

        compute(buf_b, half)
        pltpu.sync_copy(acc_v.at[pl.ds(0, nLen)], out_hbm.at[b, c])
        return _
    lax.fori_loop(0, jobs_per_worker, job, None)


def _run_sc(TRFs, sourceIdx, ov_arr, b_count):
    _, outDim, nWin, nSeq = TRFs.shape
    nLen = 2 * nSeq
    accN = nLen + nWin
    mesh = plsc.VectorSubcoreMesh(core_axis_name="c", subcore_axis_name="s")
    run = pl.kernel(
        functools.partial(_sc_body, b_count, outDim, nWin, nSeq),
        mesh=mesh,
        compiler_params=pltpu.CompilerParams(needs_layout_passes=False),
        out_type=jax.ShapeDtypeStruct((b_count, outDim, nLen), jnp.float32),
        scratch_types=[
            pltpu.VMEM((nWin, nSeq // 2), jnp.float32),
            pltpu.VMEM((nWin, nSeq // 2), jnp.float32),
            pltpu.VMEM((nSeq,), jnp.int32),
            pltpu.VMEM((_L,), jnp.float32),
            pltpu.VMEM((accN,), jnp.float32),
            pltpu.SemaphoreType.DMA,
            pltpu.SemaphoreType.DMA,
        ],
    )
    return run(TRFs, sourceIdx, ov_arr)


def _tc_body(nWin, nSeq, trf_ref, src_ref, ov_ref, oe_ref, oo_ref):
    trf = trf_ref[0]
    src = src_ref[0]
    o = src - 2 * lax.broadcasted_iota(jnp.int32, (1, nSeq), 1)
    m0 = (o == 0).astype(jnp.float32)
    m1 = 1.0 - m0
    pad = nWin // 2

    def shifted(x, left):
        parts = []
        if left:
            parts.append(jnp.zeros((_CBLK, left), jnp.float32))
        parts.append(x)
        if pad - left:
            parts.append(jnp.zeros((_CBLK, pad - left), jnp.float32))
        return jnp.concatenate(parts, axis=1) if len(parts) > 1 else x

    acc_e = jnp.zeros((_CBLK, nSeq + pad), jnp.float32)
    acc_o = jnp.zeros((_CBLK, nSeq + pad), jnp.float32)
    for w in range(nWin):
        v = trf[:, w, :]
        mE, mO = (m0, m1) if w % 2 == 0 else (m1, m0)
        acc_e = acc_e + shifted(v * mE, (w + 1) // 2)
        acc_o = acc_o + shifted(v * mO, w // 2)
    ov = ov_ref[0, 0]
    oe_ref[0] = acc_e[:, :nSeq] + ov
    oo_ref[0] = acc_o[:, :nSeq] + ov


def _run_tc(TRFs, sourceIdx, ov2, b_lo):
    nBatch, outDim, nWin, nSeq = TRFs.shape
    nTc = nBatch - b_lo
    src3 = sourceIdx.reshape(nBatch, 1, nSeq)
    grid = (nTc, outDim // _CBLK)
    oe, oo = pl.pallas_call(
        functools.partial(_tc_body, nWin, nSeq),
        grid=grid,
        in_specs=[
            pl.BlockSpec((1, _CBLK, nWin, nSeq),
                         lambda i, j: (i + b_lo, j, 0, 0)),
            pl.BlockSpec((1, 1, nSeq), lambda i, j: (i + b_lo, 0, 0)),
            pl.BlockSpec((1, 1), lambda i, j: (0, 0)),
        ],
        out_specs=[
            pl.BlockSpec((1, _CBLK, nSeq), lambda i, j: (i, j, 0)),
            pl.BlockSpec((1, _CBLK, nSeq), lambda i, j: (i, j, 0)),
        ],
        out_shape=[
            jax.ShapeDtypeStruct((nTc, outDim, nSeq), jnp.float32),
            jax.ShapeDtypeStruct((nTc, outDim, nSeq), jnp.float32),
        ],
        compiler_params=pltpu.CompilerParams(
            dimension_semantics=("parallel", "parallel")),
    )(TRFs, src3, ov2)
    return jnp.stack([oe, oo], axis=-1).reshape(nTc, outDim, 2 * nSeq)


def kernel(TRFs, sourceIdx, nRealLen):
    nBatch, outDim, nWin, nSeq = TRFs.shape

    maxSrc = jnp.max(sourceIdx[:, -1])
    overflow = jnp.maximum(maxSrc + 1 - nRealLen, 0).astype(jnp.float32)

    out_sc = _run_sc(TRFs, sourceIdx,
                     jnp.broadcast_to(overflow, (_L,)), _BSC)
    out_tc = _run_tc(TRFs, sourceIdx, overflow.reshape(1, 1), _BSC)
    return jnp.concatenate([out_sc, out_tc], axis=0)


# speedup vs baseline: 2.0186x; 1.2233x over previous
import functools

import jax
import jax.numpy as jnp
from jax import lax
from jax.experimental import pallas as pl
from jax.experimental.pallas import tpu as pltpu
from jax.experimental.pallas import tpu_sc as plsc

_L = 16
_BSC = 4
_CBLK = 16


def _sc_body(nBatch, outDim, nWin, nSeq,
             trf_hbm, src_hbm, ov_hbm, out_hbm,
             buf_a, buf_b, src_v, ov_v, acc_v, sem_a, sem_b):
    nLen = 2 * nSeq
    accN = nLen + nWin
    half = nSeq // 2
    cid = lax.axis_index("c")
    sid = lax.axis_index("s")
    wid = sid * 2 + cid
    jobs_per_worker = (nBatch * outDim) // 32
    cblocks = 32 // nBatch
    b = wid // cblocks
    c0 = (wid % cblocks) * jobs_per_worker

    pltpu.sync_copy(src_hbm.at[b], src_v)
    pltpu.sync_copy(ov_hbm, ov_v)
    ovec = ov_v[...]

    def in_copy(c, h, buf, sem):
        return pltpu.make_async_copy(
            trf_hbm.at[b, c, :, pl.ds(h * half, half)], buf, sem)

    def compute(buf, s_base):
        @plsc.parallel_loop(0, half // _L, unroll=2)
        def sgroup(sb):
            tvec = src_v[pl.ds(s_base + sb * _L, _L)]
            vals = [buf[w, pl.ds(sb * _L, _L)] for w in range(nWin)]
            idxs = [tvec + r for r in range(8)]
            for w in range(nWin):
                base = (w // 8) * 8
                plsc.addupdate_scatter(
                    acc_v.at[pl.ds(base, accN - base)], [idxs[w % 8]], vals[w])

    in_copy(c0, 0, buf_a, sem_a).start()

    def job(j, _):
        c = c0 + j

        def init(i, _):
            acc_v[pl.ds(i * _L, _L)] = ovec
            return _
        lax.fori_loop(0, accN // _L, init, None)

        in_copy(c, 0, buf_a, sem_a).wait()
        in_copy(c, 1, buf_b, sem_b).start()
        compute(buf_a, 0)
        in_copy(c, 1, buf_b, sem_b).wait()

        @pl.when(j < jobs_per_worker - 1)
        def _prefetch():
            in_copy(c + 1, 0, buf_a, sem_a).start()

        compute(buf_b, half)
        pltpu.sync_copy(acc_v.at[pl.ds(0, nLen)], out_hbm.at[b, c])
        return _
    lax.fori_loop(0, jobs_per_worker, job, None)


def _run_sc(TRFs, sourceIdx, ov_arr, b_count):
    _, outDim, nWin, nSeq = TRFs.shape
    nLen = 2 * nSeq
    accN = nLen + nWin
    mesh = plsc.VectorSubcoreMesh(core_axis_name="c", subcore_axis_name="s")
    run = pl.kernel(
        functools.partial(_sc_body, b_count, outDim, nWin, nSeq),
        mesh=mesh,
        compiler_params=pltpu.CompilerParams(needs_layout_passes=False),
        out_type=jax.ShapeDtypeStruct((b_count, outDim, nLen), jnp.float32),
        scratch_types=[
            pltpu.VMEM((nWin, nSeq // 2), jnp.float32),
            pltpu.VMEM((nWin, nSeq // 2), jnp.float32),
            pltpu.VMEM((nSeq,), jnp.int32),
            pltpu.VMEM((_L,), jnp.float32),
            pltpu.VMEM((accN,), jnp.float32),
            pltpu.SemaphoreType.DMA,
            pltpu.SemaphoreType.DMA,
        ],
    )
    return run(TRFs, sourceIdx, ov_arr)


def _tc_body(nWin, nSeq, trf_ref, src_ref, ov_ref, oe_ref, oo_ref):
    trf = trf_ref[0]
    src = src_ref[0]
    o1 = (src - 2 * lax.broadcasted_iota(jnp.int32, (1, nSeq), 1)) == 1
    b1 = jnp.broadcast_to(o1, (_CBLK, nSeq))
    pad = nWin // 2
    zrow = jnp.zeros((_CBLK, nSeq), jnp.float32)

    def shifted(x, left):
        parts = []
        if left:
            parts.append(jnp.zeros((_CBLK, left), jnp.float32))
        parts.append(x)
        if pad - left:
            parts.append(jnp.zeros((_CBLK, pad - left), jnp.float32))
        return jnp.concatenate(parts, axis=1) if len(parts) > 1 else x

    acc_e = jnp.zeros((_CBLK, nSeq + pad), jnp.float32)
    acc_o = jnp.zeros((_CBLK, nSeq + pad), jnp.float32)
    for k in range(nWin // 2 + 1):
        lo = trf[:, 2 * k - 1, :] if 2 * k - 1 >= 0 else zrow
        hi = trf[:, 2 * k, :] if 2 * k < nWin else zrow
        acc_e = acc_e + shifted(jnp.where(b1, lo, hi), k)
    for k in range(nWin // 2):
        acc_o = acc_o + shifted(
            jnp.where(b1, trf[:, 2 * k, :], trf[:, 2 * k + 1, :]), k)
    ov = ov_ref[0, 0]
    oe_ref[...] = acc_e[:, :nSeq] + ov
    oo_ref[...] = acc_o[:, :nSeq] + ov


def _run_tc(TRFs, sourceIdx, ov2, b_lo):
    nBatch, outDim, nWin, nSeq = TRFs.shape
    src3 = sourceIdx.reshape(nBatch, 1, nSeq)
    outs = []
    for b in range(b_lo, nBatch):
        oe, oo = pl.pallas_call(
            functools.partial(_tc_body, nWin, nSeq),
            grid=(outDim // _CBLK,),
            in_specs=[
                pl.BlockSpec((1, _CBLK, nWin, nSeq),
                             lambda j, b=b: (b, j, 0, 0)),
                pl.BlockSpec((1, 1, nSeq), lambda j, b=b: (b, 0, 0)),
                pl.BlockSpec((1, 1), lambda j: (0, 0)),
            ],
            out_specs=[
                pl.BlockSpec((_CBLK, nSeq), lambda j: (j, 0)),
                pl.BlockSpec((_CBLK, nSeq), lambda j: (j, 0)),
            ],
            out_shape=[
                jax.ShapeDtypeStruct((outDim, nSeq), jnp.float32),
                jax.ShapeDtypeStruct((outDim, nSeq), jnp.float32),
            ],
            compiler_params=pltpu.CompilerParams(
                dimension_semantics=("parallel",)),
        )(TRFs, src3, ov2)
        outs.append(jnp.stack([oe, oo], axis=-1).reshape(outDim, 2 * nSeq))
    return jnp.stack(outs, axis=0)


def kernel(TRFs, sourceIdx, nRealLen):
    nBatch, outDim, nWin, nSeq = TRFs.shape

    maxSrc = jnp.max(sourceIdx[:, -1])
    overflow = jnp.maximum(maxSrc + 1 - nRealLen, 0).astype(jnp.float32)

    out_sc = _run_sc(TRFs, sourceIdx,
                     jnp.broadcast_to(overflow, (_L,)), _BSC)
    out_tc = _run_tc(TRFs, sourceIdx, overflow.reshape(1, 1), _BSC)
    return jnp.concatenate([out_sc, out_tc], axis=0)
